# Initial kernel scaffold; baseline (speedup 1.0000x reference)
import jax
import jax.numpy as jnp
from jax.experimental import pallas as pl


def kernel(x, edge_index, edge_weight, W1, b1, W2, b2, W3, b3, Wxz, bxz, Whz, bhz, Wxr, bxr, Whr, bhr, Wxh, bxh, Whh, bhh, Wl, bl):
    raise NotImplementedError("write your pallas kernel here")


# speedup vs baseline: 15.2427x; 15.2427x over previous
import functools

import jax
import jax.numpy as jnp
from jax import lax
from jax.experimental import pallas as pl
from jax.experimental.pallas import tpu as pltpu
from jax.experimental.pallas import tpu_sc as plsc

N = 10000
E = 640000
NW = 32


def _wid():
    return lax.axis_index("c") * 16 + lax.axis_index("s")


def _zero_acc(acc, n):
    def zbody(i, _):
        acc[pl.ds(i * 16, 16)] = jnp.zeros((16,), jnp.float32)
        return 0
    lax.fori_loop(0, n // 16, zbody, 0)


def _make_spmm(n, e, c_pad, cpt, chunk, interpret=False):
    ncg = c_pad // cpt
    nsplit = NW // ncg
    se = e // nsplit
    nchunk = se // chunk
    groups = chunk // 16
    mesh = plsc.VectorSubcoreMesh(core_axis_name="c", subcore_axis_name="s",
                                  num_cores=2, num_subcores=16)

    scratch = ([pltpu.VMEM((n,), jnp.float32) for _ in range(cpt)]
               + [pltpu.VMEM((n,), jnp.float32) for _ in range(cpt)]
               + [pltpu.VMEM((chunk,), jnp.int32),
                  pltpu.VMEM((chunk,), jnp.int32),
                  pltpu.VMEM((chunk,), jnp.float32)])

    @functools.partial(
        pl.kernel,
        out_type=jax.ShapeDtypeStruct((nsplit * c_pad, n), jnp.float32),
        mesh=mesh,
        scratch_types=scratch,
        compiler_params=pltpu.CompilerParams(needs_layout_passes=False),
        interpret=interpret,
    )
    def spmm_kernel(gt_hbm, src_hbm, dst_hbm, w_hbm, out_hbm, *refs):
        cols = refs[:cpt]
        accs = refs[cpt:2 * cpt]
        src_v, dst_v, w_v = refs[2 * cpt:]
        wid = _wid()
        cg = wid % ncg
        split = wid // ncg
        for c in range(cpt):
            pltpu.sync_copy(gt_hbm.at[cg * cpt + c], cols[c])
            _zero_acc(accs[c], n)

        def chunk_body(ch, _):
            off = split * se + ch * chunk
            pltpu.sync_copy(src_hbm.at[pl.ds(off, chunk)], src_v)
            pltpu.sync_copy(dst_hbm.at[pl.ds(off, chunk)], dst_v)
            pltpu.sync_copy(w_hbm.at[pl.ds(off, chunk)], w_v)

            def group_body(g, _):
                base = g * 16
                s = src_v[pl.ds(base, 16)]
                d = dst_v[pl.ds(base, 16)]
                wv = w_v[pl.ds(base, 16)]
                for c in range(cpt):
                    vals = plsc.load_gather(cols[c], [s])
                    plsc.addupdate_scatter(accs[c], [d], vals * wv)
                return 0
            lax.fori_loop(0, groups, group_body, 0)
            return 0
        lax.fori_loop(0, nchunk, chunk_body, 0)
        for c in range(cpt):
            pltpu.sync_copy(accs[c], out_hbm.at[split * c_pad + cg * cpt + c])

    return spmm_kernel


def _make_deg(n, e, interpret=False):
    se = e // NW
    groups = se // 16
    mesh = plsc.VectorSubcoreMesh(core_axis_name="c", subcore_axis_name="s",
                                  num_cores=2, num_subcores=16)

    @functools.partial(
        pl.kernel,
        out_type=jax.ShapeDtypeStruct((NW, n), jnp.float32),
        mesh=mesh,
        scratch_types=[
            pltpu.VMEM((n,), jnp.float32),
            pltpu.VMEM((se,), jnp.int32),
            pltpu.VMEM((se,), jnp.float32),
        ],
        compiler_params=pltpu.CompilerParams(needs_layout_passes=False),
        interpret=interpret,
    )
    def deg_kernel(src_hbm, w_hbm, out_hbm, acc, src_v, w_v):
        wid = _wid()
        off = wid * se
        pltpu.sync_copy(src_hbm.at[pl.ds(off, se)], src_v)
        pltpu.sync_copy(w_hbm.at[pl.ds(off, se)], w_v)
        _zero_acc(acc, n)

        def group_body(g, _):
            base = g * 16
            s = src_v[pl.ds(base, 16)]
            wv = w_v[pl.ds(base, 16)]
            plsc.addupdate_scatter(acc, [s], wv)
            return 0
        lax.fori_loop(0, groups, group_body, 0)
        pltpu.sync_copy(acc, out_hbm.at[wid])

    return deg_kernel


_make_spmm = functools.cache(_make_spmm)
_make_deg = functools.cache(_make_deg)


def kernel(x, edge_index, edge_weight, W1, b1, W2, b2, W3, b3,
           Wxz, bxz, Whz, bhz, Wxr, bxr, Whr, bhr, Wxh, bxh, Whh, bhh, Wl, bl):
    n = x.shape[2]
    src = edge_index[0]
    dst = edge_index[1]
    _spmm64 = _make_spmm(N, E, 64, 4, 8000)
    _spmm32 = _make_spmm(N, E, 32, 4, 8000)
    _deg = _make_deg(N, E)

    deg = jnp.sum(_deg(src, edge_weight), axis=0)
    dis = jnp.where(deg > 0, lax.rsqrt(jnp.where(deg > 0, deg, 1.0)), 0.0)

    def spmm32(gT):
        p = _spmm32(gT, src, dst, edge_weight)
        return p.reshape(4, 32, n).sum(axis=0)

    xT = x[0].transpose(0, 2, 1)

    def tconv(W, b):
        out = b[None, :, None]
        for k in range(3):
            out = out + jnp.einsum('oc,tcn->ton', W[:, :, k], xT[k:k + 10])
        return out
    P = tconv(W1, b1)
    Q = jax.nn.sigmoid(tconv(W2, b2))
    HT = jax.nn.relu(P * Q + tconv(W3, b3))

    g64 = jnp.zeros((64, n), jnp.float32).at[:50].set(
        (HT * dis[None, None, :]).reshape(50, n))
    AH = dis[None, :] * _spmm64(g64, src, dst, edge_weight).reshape(2, 64, n).sum(axis=0)[:50]
    AHT = AH.reshape(10, 5, n)

    def proj(W, b, A_, B_):
        return (jnp.einsum('of,ton->tfn', W[0], A_)
                - jnp.einsum('of,ton->tfn', W[1], B_) + b[None, :, None])
    XzT = proj(Wxz, bxz, HT, AHT)
    XrT = proj(Wxr, bxr, HT, AHT)
    XhT = proj(Wxh, bxh, HT, AHT)

    hT = jnp.zeros((32, n), jnp.float32)
    for t in range(10):
        AhT = dis[None, :] * spmm32(dis[None, :] * hT)
        Z = jax.nn.sigmoid(XzT[t] + Whz[0].T @ hT - Whz[1].T @ AhT + bhz[:, None])
        Rg = jax.nn.sigmoid(XrT[t] + Whr[0].T @ hT - Whr[1].T @ AhT + bhr[:, None])
        hrT = hT * Rg
        AhrT = dis[None, :] * spmm32(dis[None, :] * hrT)
        Ht = jnp.tanh(XhT[t] + Whh[0].T @ hrT - Whh[1].T @ AhrT + bhh[:, None])
        hT = Z * hT + (1.0 - Z) * Ht
    h = jax.nn.relu(hT.T)
    out = h @ Wl + bl
    return jax.nn.log_softmax(out, axis=1)


# speedup vs baseline: 46.9237x; 3.0784x over previous
import functools

import jax
import jax.numpy as jnp
from jax import lax
from jax.experimental import pallas as pl
from jax.experimental.pallas import tpu as pltpu
from jax.experimental.pallas import tpu_sc as plsc

N = 10000
E = 640000
NW = 32


def _wid():
    return lax.axis_index("c") * 16 + lax.axis_index("s")


def _zero_acc(acc, n):
    def zbody(i, _):
        acc[pl.ds(i * 16, 16)] = jnp.zeros((16,), jnp.float32)
        return 0
    lax.fori_loop(0, n // 16, zbody, 0)


def _make_spmm(n, e, c_pad, cpt, chunk, interpret=False):
    ncg = c_pad // cpt
    nsplit = NW // ncg
    se = e // nsplit
    nchunk = se // chunk
    groups = chunk // 16
    mesh = plsc.VectorSubcoreMesh(core_axis_name="c", subcore_axis_name="s",
                                  num_cores=2, num_subcores=16)

    scratch = ([pltpu.VMEM((n,), jnp.float32) for _ in range(cpt)]
               + [pltpu.VMEM((n,), jnp.float32) for _ in range(cpt)]
               + [pltpu.VMEM((chunk,), jnp.int32) for _ in range(2)]
               + [pltpu.VMEM((chunk,), jnp.int32) for _ in range(2)]
               + [pltpu.VMEM((chunk,), jnp.float32) for _ in range(2)]
               + [pltpu.SemaphoreType.DMA for _ in range(6)])

    @functools.partial(
        pl.kernel,
        out_type=jax.ShapeDtypeStruct((nsplit * c_pad, n), jnp.float32),
        mesh=mesh,
        scratch_types=scratch,
        compiler_params=pltpu.CompilerParams(needs_layout_passes=False),
        interpret=interpret,
    )
    def spmm_kernel(gt_hbm, src_hbm, dst_hbm, w_hbm, out_hbm, *refs):
        cols = refs[:cpt]
        accs = refs[cpt:2 * cpt]
        src_v = refs[2 * cpt:2 * cpt + 2]
        dst_v = refs[2 * cpt + 2:2 * cpt + 4]
        w_v = refs[2 * cpt + 4:2 * cpt + 6]
        sems = refs[2 * cpt + 6:]
        wid = _wid()
        cg = wid % ncg
        split = wid // ncg
        for c in range(cpt):
            pltpu.sync_copy(gt_hbm.at[cg * cpt + c], cols[c])

            @plsc.parallel_loop(0, n // 16, unroll=5)
            def zbody(i, c=c):
                accs[c][pl.ds(i * 16, 16)] = jnp.zeros((16,), jnp.float32)

        def start_fetch(ch):
            b = ch % 2
            off = split * se + ch * chunk
            return (pltpu.async_copy(src_hbm.at[pl.ds(off, chunk)], src_v[b], sems[3 * b]),
                    pltpu.async_copy(dst_hbm.at[pl.ds(off, chunk)], dst_v[b], sems[3 * b + 1]),
                    pltpu.async_copy(w_hbm.at[pl.ds(off, chunk)], w_v[b], sems[3 * b + 2]))

        pend = start_fetch(0)
        for ch in range(nchunk):
            b = ch % 2
            for p in pend:
                p.wait()
            if ch + 1 < nchunk:
                pend = start_fetch(ch + 1)

            @plsc.parallel_loop(0, groups, unroll=4)
            def group_body(g, b=b):
                base = g * 16
                s = src_v[b][pl.ds(base, 16)]
                d = dst_v[b][pl.ds(base, 16)]
                wv = w_v[b][pl.ds(base, 16)]
                for c in range(cpt):
                    vals = plsc.load_gather(cols[c], [s])
                    plsc.addupdate_scatter(accs[c], [d], vals * wv)

        for c in range(cpt):
            pltpu.sync_copy(accs[c], out_hbm.at[split * c_pad + cg * cpt + c])

    return spmm_kernel


def _make_deg(n, e, interpret=False):
    se = e // NW
    groups = se // 16
    mesh = plsc.VectorSubcoreMesh(core_axis_name="c", subcore_axis_name="s",
                                  num_cores=2, num_subcores=16)

    @functools.partial(
        pl.kernel,
        out_type=jax.ShapeDtypeStruct((NW, n), jnp.float32),
        mesh=mesh,
        scratch_types=[
            pltpu.VMEM((n,), jnp.float32),
            pltpu.VMEM((se,), jnp.int32),
            pltpu.VMEM((se,), jnp.float32),
        ],
        compiler_params=pltpu.CompilerParams(needs_layout_passes=False),
        interpret=interpret,
    )
    def deg_kernel(src_hbm, w_hbm, out_hbm, acc, src_v, w_v):
        wid = _wid()
        off = wid * se
        pltpu.sync_copy(src_hbm.at[pl.ds(off, se)], src_v)
        pltpu.sync_copy(w_hbm.at[pl.ds(off, se)], w_v)
        _zero_acc(acc, n)

        def group_body(g, _):
            base = g * 16
            s = src_v[pl.ds(base, 16)]
            wv = w_v[pl.ds(base, 16)]
            plsc.addupdate_scatter(acc, [s], wv)
            return 0
        lax.fori_loop(0, groups, group_body, 0)
        pltpu.sync_copy(acc, out_hbm.at[wid])

    return deg_kernel


_make_spmm = functools.cache(_make_spmm)
_make_deg = functools.cache(_make_deg)


def kernel(x, edge_index, edge_weight, W1, b1, W2, b2, W3, b3,
           Wxz, bxz, Whz, bhz, Wxr, bxr, Whr, bhr, Wxh, bxh, Whh, bhh, Wl, bl):
    n = x.shape[2]
    src = edge_index[0]
    dst = edge_index[1]
    _spmm64 = _make_spmm(N, E, 64, 4, 8000)
    _spmm32 = _make_spmm(N, E, 32, 4, 8000)
    _deg = _make_deg(N, E)

    deg = jnp.sum(_deg(src, edge_weight), axis=0)
    dis = jnp.where(deg > 0, lax.rsqrt(jnp.where(deg > 0, deg, 1.0)), 0.0)

    def spmm32(gT):
        p = _spmm32(gT, src, dst, edge_weight)
        return p.reshape(4, 32, n).sum(axis=0)

    xT = x[0].transpose(0, 2, 1)

    def tconv(W, b):
        out = b[None, :, None]
        for k in range(3):
            out = out + jnp.einsum('oc,tcn->ton', W[:, :, k], xT[k:k + 10])
        return out
    P = tconv(W1, b1)
    Q = jax.nn.sigmoid(tconv(W2, b2))
    HT = jax.nn.relu(P * Q + tconv(W3, b3))

    g64 = jnp.zeros((64, n), jnp.float32).at[:50].set(
        (HT * dis[None, None, :]).reshape(50, n))
    AH = dis[None, :] * _spmm64(g64, src, dst, edge_weight).reshape(2, 64, n).sum(axis=0)[:50]
    AHT = AH.reshape(10, 5, n)

    def proj(W, b, A_, B_):
        return (jnp.einsum('of,ton->tfn', W[0], A_)
                - jnp.einsum('of,ton->tfn', W[1], B_) + b[None, :, None])
    XzT = proj(Wxz, bxz, HT, AHT)
    XrT = proj(Wxr, bxr, HT, AHT)
    XhT = proj(Wxh, bxh, HT, AHT)

    hT = jnp.zeros((32, n), jnp.float32)
    for t in range(10):
        AhT = dis[None, :] * spmm32(dis[None, :] * hT)
        Z = jax.nn.sigmoid(XzT[t] + Whz[0].T @ hT - Whz[1].T @ AhT + bhz[:, None])
        Rg = jax.nn.sigmoid(XrT[t] + Whr[0].T @ hT - Whr[1].T @ AhT + bhr[:, None])
        hrT = hT * Rg
        AhrT = dis[None, :] * spmm32(dis[None, :] * hrT)
        Ht = jnp.tanh(XhT[t] + Whh[0].T @ hrT - Whh[1].T @ AhrT + bhh[:, None])
        hT = Z * hT + (1.0 - Z) * Ht
    h = jax.nn.relu(hT.T)
    out = h @ Wl + bl
    return jax.nn.log_softmax(out, axis=1)
